# baseline (speedup 1.0000x reference)
import jax
import jax.numpy as jnp
from jax.experimental import pallas as pl
from jax.experimental.pallas import tpu as pltpu

_LOOP_ITERS = 7
_BLOCK_ROWS = 16
_HALF = 8
_CH = 1024


def _solve(tau, lo, hi, F, H, n):
    below = F >= 1.0
    lo = jnp.where(below, tau, lo)
    hi = jnp.where(below, hi, tau)
    disc = H * H - n * (F - 1.0)
    tq = tau + (H - jnp.sqrt(jnp.maximum(disc, 0.0))) / jnp.maximum(n, 1.0)
    ok = (disc >= 0.0) & (n > 0.0) & (tq >= lo) & (tq <= hi)
    tau = jnp.where(ok, tq, (lo + hi) * 0.5)
    return tau, lo, hi


def _estimate_n(tau, tau_p, H, H_p, n_p):
    dt = tau - tau_p
    n_est = (H_p - H) / jnp.where(dt == 0.0, 1.0, dt)
    use = (jnp.abs(dt) >= 3e-5) & (n_est > 0.0)
    return jnp.where(use, n_est, n_p)


def _sweep(xs_ref, rows, tau):
    L = xs_ref.shape[1]
    fa = jnp.zeros((_HALF, _CH), jnp.float32)
    ha = jnp.zeros((_HALF, _CH), jnp.float32)
    for c in range(0, L, _CH):
        r = jnp.maximum(xs_ref[rows, c:c + _CH] - tau, 0.0)
        ha = ha + r
        fa = fa + r * r
    F = jnp.sum(fa, axis=1, keepdims=True)
    H = jnp.sum(ha, axis=1, keepdims=True)
    return F, H


def _tsallis_block(x_ref, o_ref, xs_ref):
    L = x_ref.shape[1]
    rows_a = slice(0, _HALF)
    rows_b = slice(_HALF, _BLOCK_ROWS)

    macc = x_ref[:, 0:_CH]
    for c in range(_CH, L, _CH):
        macc = jnp.maximum(macc, x_ref[:, c:c + _CH])
    maxv = jnp.max(macc, axis=1, keepdims=True)

    def fused_sweep(rows, mv):
        fa = jnp.zeros((_HALF, _CH), jnp.float32)
        ha = jnp.zeros_like(fa)
        na = jnp.zeros_like(fa)
        for c in range(0, L, _CH):
            xs = (x_ref[rows, c:c + _CH] - mv) * 0.5
            xs_ref[rows, c:c + _CH] = xs
            r = jnp.maximum(xs + 1.0, 0.0)
            ha = ha + r
            fa = fa + r * r
            na = na + jnp.where(r > 0.0, 1.0, 0.0)
        return (jnp.sum(fa, axis=1, keepdims=True),
                jnp.sum(ha, axis=1, keepdims=True),
                jnp.sum(na, axis=1, keepdims=True))

    Fa0, Ha0, na0 = fused_sweep(rows_a, maxv[rows_a])
    Fb0, Hb0, nb0 = fused_sweep(rows_b, maxv[rows_b])

    lo0 = jnp.full((_HALF, 1), -1.0, jnp.float32)
    hi0 = jnp.zeros((_HALF, 1), jnp.float32)

    za = Fa0 * 0.0
    zb = Fb0 * 0.0
    ta, la, ha_ = _solve(lo0 + za, lo0 + za, hi0 + za, Fa0, Ha0, na0)
    sa = (ta, lo0 + za, Ha0, na0, la, ha_)
    sb = (lo0 + zb, lo0 + zb, Hb0, nb0, lo0 + zb, hi0 + zb, Fb0, Hb0)

    def body(_, carry):
        (ta, tpa, hpa, npa, la, hia), (tb, tpb, hpb, npb, lb, hib, fb, hb) = carry
        nb = _estimate_n(tb, tpb, hb, hpb, npb)
        tb_new, lb, hib = _solve(tb, lb, hib, fb, hb, nb)
        Fa, Ha = _sweep(xs_ref, rows_a, ta)
        na_ = _estimate_n(ta, tpa, Ha, hpa, npa)
        ta_new, la, hia = _solve(ta, la, hia, Fa, Ha, na_)
        Fb, Hb = _sweep(xs_ref, rows_b, tb_new)
        return ((ta_new, ta, Ha, na_, la, hia),
                (tb_new, tb, hb, nb, lb, hib, Fb, Hb))

    sa, sb = jax.lax.fori_loop(0, _LOOP_ITERS, body, (sa, sb))
    ta = sa[0]
    (tb, tpb, hpb, npb, lb, hib, fb, hb) = sb
    nb = _estimate_n(tb, tpb, hb, hpb, npb)
    tb, _, _ = _solve(tb, lb, hib, fb, hb, nb)

    for c in range(0, L, _CH):
        r = jnp.maximum(xs_ref[rows_a, c:c + _CH] - ta, 0.0)
        o_ref[rows_a, c:c + _CH] = r * r
    for c in range(0, L, _CH):
        r = jnp.maximum(xs_ref[rows_b, c:c + _CH] - tb, 0.0)
        o_ref[rows_b, c:c + _CH] = r * r


def kernel(X):
    R, L = X.shape
    return pl.pallas_call(
        _tsallis_block,
        grid=(R // _BLOCK_ROWS,),
        in_specs=[pl.BlockSpec((_BLOCK_ROWS, L), lambda i: (i, 0))],
        out_specs=pl.BlockSpec((_BLOCK_ROWS, L), lambda i: (i, 0)),
        out_shape=jax.ShapeDtypeStruct((R, L), jnp.float32),
        scratch_shapes=[pltpu.VMEM((_BLOCK_ROWS, L), jnp.float32)],
        compiler_params=pltpu.CompilerParams(
            dimension_semantics=("parallel",)),
    )(X)
